# Initial kernel scaffold; baseline (speedup 1.0000x reference)
import jax
import jax.numpy as jnp
from jax.experimental import pallas as pl


def kernel(pred_mask, positive_points, negative_points):
    raise NotImplementedError("write your pallas kernel here")


# speedup vs baseline: 1.5083x; 1.5083x over previous
import functools

import jax
import jax.numpy as jnp
from jax import lax
from jax.experimental import pallas as pl
from jax.experimental.pallas import tpu as pltpu
from jax.experimental.pallas import tpu_sc as plsc

B, H, W = 16, 512, 512
NPOS, NNEG = 256, 256
PTS_PER_B = NPOS + NNEG
P = B * PTS_PER_B
NW = 32
PPW = P // NW
LANES = 16
GCHUNK = 128


def _sc_sample_kernel():
    mesh = plsc.VectorSubcoreMesh(core_axis_name="c", subcore_axis_name="s")

    @functools.partial(
        pl.kernel,
        mesh=mesh,
        out_type=jax.ShapeDtypeStruct((P,), jnp.float32),
        scratch_types=[
            pltpu.VMEM((PPW,), jnp.int32),
            pltpu.VMEM((PPW,), jnp.int32),
            pltpu.VMEM((4 * PPW,), jnp.int32),
            pltpu.VMEM((4 * PPW,), jnp.float32),
            pltpu.VMEM((4 * PPW,), jnp.float32),
            pltpu.VMEM((PPW,), jnp.float32),
            pltpu.SemaphoreType.DMA,
        ],
    )
    def sample(mask_hbm, xs_hbm, ys_hbm, out_hbm, xv, yv, idxv, wv, valv, sv, sem):
        c = lax.axis_index("c")
        s = lax.axis_index("s")
        wid = s * 2 + c
        base_pt = wid * PPW
        mask_base = (wid // 2) * (H * W)

        pltpu.sync_copy(xs_hbm.at[pl.ds(base_pt, PPW)], xv)
        pltpu.sync_copy(ys_hbm.at[pl.ds(base_pt, PPW)], yv)

        for j in range(PPW // LANES):
            sl = pl.ds(j * LANES, LANES)
            xf = xv[sl].astype(jnp.float32)
            yf = yv[sl].astype(jnp.float32)
            xn = 2.0 * xf / float(W - 1) - 1.0
            yn = 2.0 * yf / float(H - 1) - 1.0
            ix = (xn + 1.0) / 2.0 * float(W - 1)
            iy = (yn + 1.0) / 2.0 * float(H - 1)
            x0 = ix.astype(jnp.int32)
            y0 = iy.astype(jnp.int32)
            wx1 = ix - x0.astype(jnp.float32)
            wy1 = iy - y0.astype(jnp.float32)
            wx0 = 1.0 - wx1
            wy0 = 1.0 - wy1
            x1 = x0 + 1
            y1 = y0 + 1
            in_x1 = x1 <= W - 1
            in_y1 = y1 <= H - 1
            x1c = jnp.minimum(x1, W - 1)
            y1c = jnp.minimum(y1, H - 1)
            row0 = mask_base + y0 * W
            row1 = mask_base + y1c * W
            idxv[pl.ds(0 * PPW + j * LANES, LANES)] = row0 + x0
            idxv[pl.ds(1 * PPW + j * LANES, LANES)] = row0 + x1c
            idxv[pl.ds(2 * PPW + j * LANES, LANES)] = row1 + x0
            idxv[pl.ds(3 * PPW + j * LANES, LANES)] = row1 + x1c
            zero = jnp.zeros((LANES,), jnp.float32)
            wv[pl.ds(0 * PPW + j * LANES, LANES)] = wy0 * wx0
            wv[pl.ds(1 * PPW + j * LANES, LANES)] = jnp.where(in_x1, wy0 * wx1, zero)
            wv[pl.ds(2 * PPW + j * LANES, LANES)] = jnp.where(in_y1, wy1 * wx0, zero)
            wv[pl.ds(3 * PPW + j * LANES, LANES)] = jnp.where(
                in_x1 & in_y1, wy1 * wx1, zero)

        copies = []
        for k in range(4 * PPW // GCHUNK):
            gsl = pl.ds(k * GCHUNK, GCHUNK)
            copies.append(
                pltpu.async_copy(mask_hbm.at[idxv.at[gsl]], valv.at[gsl], sem))
        for cp in copies:
            cp.wait()

        for j in range(PPW // LANES):
            acc = jnp.zeros((LANES,), jnp.float32)
            for corner in range(4):
                csl = pl.ds(corner * PPW + j * LANES, LANES)
                acc = acc + valv[csl] * wv[csl]
            sv[pl.ds(j * LANES, LANES)] = acc

        pltpu.sync_copy(sv, out_hbm.at[pl.ds(base_pt, PPW)])

    return sample


_sc_sample = _sc_sample_kernel()


def _bce_mean_body(s_ref, o_ref):
    s = s_ref[...]
    col = lax.broadcasted_iota(jnp.int32, (B, PTS_PER_B), 1)
    tgt = jnp.where(col < NPOS, 1.0, 0.0)
    bce = jnp.maximum(s, 0.0) - s * tgt + jnp.log1p(jnp.exp(-jnp.abs(s)))
    o_ref[...] = (jnp.sum(bce) * (1.0 / float(P))).reshape(1, 1)


def kernel(pred_mask, positive_points, negative_points):
    mask_flat = pred_mask.reshape(-1)
    pts = jnp.concatenate([positive_points, negative_points], axis=1)
    xs = pts[:, :, 0].reshape(-1).astype(jnp.int32)
    ys = pts[:, :, 1].reshape(-1).astype(jnp.int32)

    samples = _sc_sample(mask_flat, xs, ys)

    loss = pl.pallas_call(
        _bce_mean_body,
        out_shape=jax.ShapeDtypeStruct((1, 1), jnp.float32),
    )(samples.reshape(B, PTS_PER_B))
    return loss[0, 0]


# speedup vs baseline: 2.4794x; 1.6438x over previous
import functools

import jax
import jax.numpy as jnp
from jax import lax
from jax.experimental import pallas as pl
from jax.experimental.pallas import tpu as pltpu
from jax.experimental.pallas import tpu_sc as plsc

B, H, W = 16, 512, 512
NPOS, NNEG = 256, 256
PTS_PER_B = NPOS + NNEG
P = B * PTS_PER_B
NW = 32
PPW = P // NW
LANES = 16
GCHUNK = 128


def _sc_sample_kernel():
    mesh = plsc.VectorSubcoreMesh(core_axis_name="c", subcore_axis_name="s")

    @functools.partial(
        pl.kernel,
        mesh=mesh,
        out_type=jax.ShapeDtypeStruct((P,), jnp.float32),
        scratch_types=[
            pltpu.VMEM((PPW,), jnp.int32),
            pltpu.VMEM((PPW,), jnp.int32),
            pltpu.VMEM((4 * PPW,), jnp.int32),
            pltpu.VMEM((4 * PPW,), jnp.float32),
            pltpu.VMEM((4 * PPW,), jnp.float32),
            pltpu.VMEM((PPW,), jnp.float32),
            pltpu.SemaphoreType.DMA,
        ],
    )
    def sample(mask_hbm, xs_hbm, ys_hbm, out_hbm, xv, yv, idxv, wv, valv, sv, sem):
        c = lax.axis_index("c")
        s = lax.axis_index("s")
        wid = s * 2 + c
        base_pt = wid * PPW
        mask_base = (wid // 2) * (H * W)

        pltpu.sync_copy(xs_hbm.at[pl.ds(base_pt, PPW)], xv)
        pltpu.sync_copy(ys_hbm.at[pl.ds(base_pt, PPW)], yv)

        for j in range(PPW // LANES):
            sl = pl.ds(j * LANES, LANES)
            xf = xv[sl].astype(jnp.float32)
            yf = yv[sl].astype(jnp.float32)
            xn = 2.0 * xf / float(W - 1) - 1.0
            yn = 2.0 * yf / float(H - 1) - 1.0
            ix = (xn + 1.0) / 2.0 * float(W - 1)
            iy = (yn + 1.0) / 2.0 * float(H - 1)
            x0 = ix.astype(jnp.int32)
            y0 = iy.astype(jnp.int32)
            wx1 = ix - x0.astype(jnp.float32)
            wy1 = iy - y0.astype(jnp.float32)
            wx0 = 1.0 - wx1
            wy0 = 1.0 - wy1
            x1 = x0 + 1
            y1 = y0 + 1
            in_x1 = x1 <= W - 1
            in_y1 = y1 <= H - 1
            x1c = jnp.minimum(x1, W - 1)
            y1c = jnp.minimum(y1, H - 1)
            y0t = mask_base + ((y0 >> 3) << 12) + ((y0 & 7) << 7)
            y1t = mask_base + ((y1c >> 3) << 12) + ((y1c & 7) << 7)
            x0t = ((x0 >> 7) << 10) + (x0 & 127)
            x1t = ((x1c >> 7) << 10) + (x1c & 127)
            idxv[pl.ds(0 * PPW + j * LANES, LANES)] = y0t + x0t
            idxv[pl.ds(1 * PPW + j * LANES, LANES)] = y0t + x1t
            idxv[pl.ds(2 * PPW + j * LANES, LANES)] = y1t + x0t
            idxv[pl.ds(3 * PPW + j * LANES, LANES)] = y1t + x1t
            zero = jnp.zeros((LANES,), jnp.float32)
            wv[pl.ds(0 * PPW + j * LANES, LANES)] = wy0 * wx0
            wv[pl.ds(1 * PPW + j * LANES, LANES)] = jnp.where(in_x1, wy0 * wx1, zero)
            wv[pl.ds(2 * PPW + j * LANES, LANES)] = jnp.where(in_y1, wy1 * wx0, zero)
            wv[pl.ds(3 * PPW + j * LANES, LANES)] = jnp.where(
                in_x1 & in_y1, wy1 * wx1, zero)

        copies = []
        for k in range(4 * PPW // GCHUNK):
            gsl = pl.ds(k * GCHUNK, GCHUNK)
            copies.append(
                pltpu.async_copy(mask_hbm.at[idxv.at[gsl]], valv.at[gsl], sem))
        for cp in copies:
            cp.wait()

        for j in range(PPW // LANES):
            acc = jnp.zeros((LANES,), jnp.float32)
            for corner in range(4):
                csl = pl.ds(corner * PPW + j * LANES, LANES)
                acc = acc + valv[csl] * wv[csl]
            sv[pl.ds(j * LANES, LANES)] = acc

        pltpu.sync_copy(sv, out_hbm.at[pl.ds(base_pt, PPW)])

    return sample


_sc_sample = _sc_sample_kernel()


def _bce_mean_body(s_ref, o_ref):
    s = s_ref[...]
    col = lax.broadcasted_iota(jnp.int32, (B, PTS_PER_B), 1)
    tgt = jnp.where(col < NPOS, 1.0, 0.0)
    bce = jnp.maximum(s, 0.0) - s * tgt + jnp.log1p(jnp.exp(-jnp.abs(s)))
    o_ref[...] = (jnp.sum(bce) * (1.0 / float(P))).reshape(1, 1)


def kernel(pred_mask, positive_points, negative_points):
    mask_flat = (
        pred_mask.reshape(B, 1, H // 8, 8, W // 128, 128)
        .transpose(0, 1, 2, 4, 3, 5)
        .reshape(-1)
    )
    pts = jnp.concatenate([positive_points, negative_points], axis=1)
    xs = pts[:, :, 0].reshape(-1).astype(jnp.int32)
    ys = pts[:, :, 1].reshape(-1).astype(jnp.int32)

    samples = _sc_sample(mask_flat, xs, ys)

    loss = pl.pallas_call(
        _bce_mean_body,
        out_shape=jax.ShapeDtypeStruct((1, 1), jnp.float32),
    )(samples.reshape(B, PTS_PER_B))
    return loss[0, 0]


# speedup vs baseline: 2.6434x; 1.0662x over previous
import functools

import jax
import jax.numpy as jnp
from jax import lax
from jax.experimental import pallas as pl
from jax.experimental.pallas import tpu as pltpu
from jax.experimental.pallas import tpu_sc as plsc

B, H, W = 16, 512, 512
NPOS, NNEG = 256, 256
PTS_PER_B = NPOS + NNEG
P = B * PTS_PER_B
NW = 32
PPW = P // NW
LANES = 16
GCHUNK = 128


def _sc_sample_kernel():
    mesh = plsc.VectorSubcoreMesh(core_axis_name="c", subcore_axis_name="s")

    @functools.partial(
        pl.kernel,
        mesh=mesh,
        out_type=jax.ShapeDtypeStruct((P,), jnp.float32),
        scratch_types=[
            pltpu.VMEM((PPW,), jnp.int32),
            pltpu.VMEM((4 * PPW,), jnp.int32),
            pltpu.VMEM((4 * PPW,), jnp.float32),
            pltpu.VMEM((4 * PPW,), jnp.float32),
            pltpu.VMEM((PPW,), jnp.float32),
            pltpu.SemaphoreType.DMA,
        ],
    )
    def sample(mask_hbm, pos_hbm, neg_hbm, out_hbm, ptv, idxv, wv, valv, sv, sem):
        c = lax.axis_index("c")
        s = lax.axis_index("s")
        wid = s * 2 + c
        base_pt = wid * PPW
        batch = wid // 2
        half = wid % 2
        mask_base = batch * (H * W)
        coord_base = batch * PPW

        @pl.when(half == 0)
        def _():
            pltpu.sync_copy(pos_hbm.at[pl.ds(coord_base, PPW)], ptv)

        @pl.when(half == 1)
        def _():
            pltpu.sync_copy(neg_hbm.at[pl.ds(coord_base, PPW)], ptv)

        for j in range(PPW // LANES):
            v = ptv[pl.ds(j * LANES, LANES)]
            x = v & 0xFFFF
            y = v >> 16
            xf = x.astype(jnp.float32)
            yf = y.astype(jnp.float32)
            xn = 2.0 * xf / float(W - 1) - 1.0
            yn = 2.0 * yf / float(H - 1) - 1.0
            ix = (xn + 1.0) / 2.0 * float(W - 1)
            iy = (yn + 1.0) / 2.0 * float(H - 1)
            x0 = ix.astype(jnp.int32)
            y0 = iy.astype(jnp.int32)
            wx1 = ix - x0.astype(jnp.float32)
            wy1 = iy - y0.astype(jnp.float32)
            wx0 = 1.0 - wx1
            wy0 = 1.0 - wy1
            x1 = x0 + 1
            y1 = y0 + 1
            in_x1 = x1 <= W - 1
            in_y1 = y1 <= H - 1
            x1c = jnp.minimum(x1, W - 1)
            y1c = jnp.minimum(y1, H - 1)
            y0t = mask_base + ((y0 >> 3) << 12) + ((y0 & 7) << 7)
            y1t = mask_base + ((y1c >> 3) << 12) + ((y1c & 7) << 7)
            x0t = ((x0 >> 7) << 10) + (x0 & 127)
            x1t = ((x1c >> 7) << 10) + (x1c & 127)
            idxv[pl.ds(0 * PPW + j * LANES, LANES)] = y0t + x0t
            idxv[pl.ds(1 * PPW + j * LANES, LANES)] = y0t + x1t
            idxv[pl.ds(2 * PPW + j * LANES, LANES)] = y1t + x0t
            idxv[pl.ds(3 * PPW + j * LANES, LANES)] = y1t + x1t
            zero = jnp.zeros((LANES,), jnp.float32)
            wv[pl.ds(0 * PPW + j * LANES, LANES)] = wy0 * wx0
            wv[pl.ds(1 * PPW + j * LANES, LANES)] = jnp.where(in_x1, wy0 * wx1, zero)
            wv[pl.ds(2 * PPW + j * LANES, LANES)] = jnp.where(in_y1, wy1 * wx0, zero)
            wv[pl.ds(3 * PPW + j * LANES, LANES)] = jnp.where(
                in_x1 & in_y1, wy1 * wx1, zero)

        copies = []
        for k in range(4 * PPW // GCHUNK):
            gsl = pl.ds(k * GCHUNK, GCHUNK)
            copies.append(
                pltpu.async_copy(mask_hbm.at[idxv.at[gsl]], valv.at[gsl], sem))
        for cp in copies:
            cp.wait()

        for j in range(PPW // LANES):
            acc = jnp.zeros((LANES,), jnp.float32)
            for corner in range(4):
                csl = pl.ds(corner * PPW + j * LANES, LANES)
                acc = acc + valv[csl] * wv[csl]
            sv[pl.ds(j * LANES, LANES)] = acc

        pltpu.sync_copy(sv, out_hbm.at[pl.ds(base_pt, PPW)])

    return sample


_sc_sample = _sc_sample_kernel()

_ROWS, _COLS = P // 128, 128


def _bce_mean_body(s_ref, o_ref):
    s = s_ref[...]
    row = lax.broadcasted_iota(jnp.int32, (_ROWS, _COLS), 0)
    tgt = jnp.where((row & 3) < 2, 1.0, 0.0)
    bce = jnp.maximum(s, 0.0) - s * tgt + jnp.log1p(jnp.exp(-jnp.abs(s)))
    o_ref[...] = (jnp.sum(bce) * (1.0 / float(P))).reshape(1, 1)


def kernel(pred_mask, positive_points, negative_points):
    mask_flat = (
        pred_mask.reshape(B, 1, H // 8, 8, W // 128, 128)
        .transpose(0, 1, 2, 4, 3, 5)
        .reshape(-1)
    )
    pp = positive_points.astype(jnp.int32)
    np_ = negative_points.astype(jnp.int32)
    pos_flat = (pp[:, :, 0] | (pp[:, :, 1] << 16)).reshape(-1)
    neg_flat = (np_[:, :, 0] | (np_[:, :, 1] << 16)).reshape(-1)

    samples = _sc_sample(mask_flat, pos_flat, neg_flat)

    loss = pl.pallas_call(
        _bce_mean_body,
        out_shape=jax.ShapeDtypeStruct((1, 1), jnp.float32),
    )(samples.reshape(_ROWS, _COLS))
    return loss[0, 0]


# speedup vs baseline: 2.8784x; 1.0889x over previous
import functools

import jax
import jax.numpy as jnp
from jax import lax
from jax.experimental import pallas as pl
from jax.experimental.pallas import tpu as pltpu
from jax.experimental.pallas import tpu_sc as plsc

B, H, W = 16, 512, 512
NPOS, NNEG = 256, 256
PTS_PER_B = NPOS + NNEG
P = B * PTS_PER_B
NW = 32
PPW = P // NW
LANES = 16
GCHUNK = 128


def _sc_sample_kernel():
    mesh = plsc.VectorSubcoreMesh(core_axis_name="c", subcore_axis_name="s")

    @functools.partial(
        pl.kernel,
        mesh=mesh,
        out_type=jax.ShapeDtypeStruct((P,), jnp.float32),
        scratch_types=[
            pltpu.VMEM((PPW,), jnp.int32),
            pltpu.VMEM((PPW,), jnp.int32),
            pltpu.VMEM((PPW,), jnp.float32),
            pltpu.SemaphoreType.DMA,
        ],
    )
    def sample(mask_hbm, pos_hbm, neg_hbm, out_hbm, ptv, idxv, sv, sem):
        c = lax.axis_index("c")
        s = lax.axis_index("s")
        wid = s * 2 + c
        base_pt = wid * PPW
        batch = wid // 2
        half = wid % 2
        mask_base = batch * (H * W)
        coord_base = batch * PPW

        @pl.when(half == 0)
        def _():
            pltpu.sync_copy(pos_hbm.at[pl.ds(coord_base, PPW)], ptv)

        @pl.when(half == 1)
        def _():
            pltpu.sync_copy(neg_hbm.at[pl.ds(coord_base, PPW)], ptv)

        for j in range(PPW // LANES):
            v = ptv[pl.ds(j * LANES, LANES)]
            x = v & 0xFFFF
            y = v >> 16
            idxv[pl.ds(j * LANES, LANES)] = (
                mask_base + ((y >> 3) << 12) + ((y & 7) << 7)
                + ((x >> 7) << 10) + (x & 127))

        copies = []
        for k in range(PPW // GCHUNK):
            gsl = pl.ds(k * GCHUNK, GCHUNK)
            copies.append(
                pltpu.async_copy(mask_hbm.at[idxv.at[gsl]], sv.at[gsl], sem))
        for cp in copies:
            cp.wait()

        pltpu.sync_copy(sv, out_hbm.at[pl.ds(base_pt, PPW)])

    return sample


_sc_sample = _sc_sample_kernel()

_ROWS, _COLS = P // 128, 128


def _bce_mean_body(s_ref, o_ref):
    s = s_ref[...]
    row = lax.broadcasted_iota(jnp.int32, (_ROWS, _COLS), 0)
    tgt = jnp.where((row & 3) < 2, 1.0, 0.0)
    bce = jnp.maximum(s, 0.0) - s * tgt + jnp.log1p(jnp.exp(-jnp.abs(s)))
    o_ref[...] = (jnp.sum(bce) * (1.0 / float(P))).reshape(1, 1)


def kernel(pred_mask, positive_points, negative_points):
    mask_flat = (
        pred_mask.reshape(B, 1, H // 8, 8, W // 128, 128)
        .transpose(0, 1, 2, 4, 3, 5)
        .reshape(-1)
    )
    pp = positive_points.astype(jnp.int32)
    np_ = negative_points.astype(jnp.int32)
    pos_flat = (pp[:, :, 0] | (pp[:, :, 1] << 16)).reshape(-1)
    neg_flat = (np_[:, :, 0] | (np_[:, :, 1] << 16)).reshape(-1)

    samples = _sc_sample(mask_flat, pos_flat, neg_flat)

    loss = pl.pallas_call(
        _bce_mean_body,
        out_shape=jax.ShapeDtypeStruct((1, 1), jnp.float32),
    )(samples.reshape(_ROWS, _COLS))
    return loss[0, 0]
